# Initial kernel scaffold; baseline (speedup 1.0000x reference)
import jax
import jax.numpy as jnp
from jax.experimental import pallas as pl


def kernel(x, edge_index, W1, attn_l1, attn_r1, b1, W2, attn_l2, attn_r2, b2):
    raise NotImplementedError("write your pallas kernel here")


# speedup vs baseline: 35.8614x; 35.8614x over previous
import functools

import numpy as np
import jax
import jax.numpy as jnp
from jax import lax
from jax.experimental import pallas as pl
from jax.experimental.pallas import tpu as pltpu
from jax.experimental.pallas import tpu_sc as plsc

N_NODES = 10000
HID = 128
H = 8
NC = 2
NS = 16
NW = NC * NS
CHUNK = 128
N_PAD = 10112
ROWS_PER_SUB = N_PAD // NS
ROW_SPLITS = (128, 128, 128, 128, 120)

_c = np.arange(HID)
SRC_COLS = ((_c % 8) * 16 + 2 * (_c // 16) + ((_c % 16) // 8)).astype(np.int32)
_PINV = np.zeros((HID, HID), np.float32)
_PINV[np.arange(HID), SRC_COLS] = 1.0
HSEL = (np.arange(8)[None, :] == (_c % 8)[:, None]).astype(np.float32)

_ROW_BLK = 1000
_GRID = N_NODES // _ROW_BLK


def _tc_pre_body(x_ref, w_ref, m_ref, f_ref, alr_ref):
    f = jnp.dot(x_ref[...], w_ref[...], preferred_element_type=jnp.float32,
                precision=lax.Precision.HIGHEST)
    f_ref[...] = f
    alr_ref[...] = jnp.dot(f, m_ref[...], preferred_element_type=jnp.float32,
                           precision=lax.Precision.HIGHEST)


def _tc_pre(x, wp, m):
    return pl.pallas_call(
        _tc_pre_body,
        grid=(_GRID,),
        in_specs=[
            pl.BlockSpec((_ROW_BLK, HID), lambda i: (i, 0)),
            pl.BlockSpec((HID, HID), lambda i: (0, 0)),
            pl.BlockSpec((HID, 16), lambda i: (0, 0)),
        ],
        out_specs=[
            pl.BlockSpec((_ROW_BLK, HID), lambda i: (i, 0)),
            pl.BlockSpec((_ROW_BLK, 16), lambda i: (i, 0)),
        ],
        out_shape=[
            jax.ShapeDtypeStruct((N_NODES, HID), jnp.float32),
            jax.ShapeDtypeStruct((N_NODES, 16), jnp.float32),
        ],
    )(x, wp, m)


def _combine(op_ref, dp_ref, b_ref):
    o = op_ref[0] + op_ref[1]
    d = dp_ref[0] + dp_ref[1]
    dt = jnp.concatenate([d] * 8, axis=1) + 1e-9
    return jnp.maximum(o / dt + b_ref[...], 0.0)


def _tc_mid_body(op_ref, dp_ref, b_ref, w_ref, m_ref, f_ref, alr_ref):
    h = _combine(op_ref, dp_ref, b_ref)
    f = jnp.dot(h, w_ref[...], preferred_element_type=jnp.float32,
                precision=lax.Precision.HIGHEST)
    f_ref[...] = f
    alr_ref[...] = jnp.dot(f, m_ref[...], preferred_element_type=jnp.float32,
                           precision=lax.Precision.HIGHEST)


def _tc_mid(out_p, den_p, bp, wp, m):
    return pl.pallas_call(
        _tc_mid_body,
        grid=(_GRID,),
        in_specs=[
            pl.BlockSpec((2, _ROW_BLK, HID), lambda i: (0, i, 0)),
            pl.BlockSpec((2, _ROW_BLK, 16), lambda i: (0, i, 0)),
            pl.BlockSpec((1, HID), lambda i: (0, 0)),
            pl.BlockSpec((HID, HID), lambda i: (0, 0)),
            pl.BlockSpec((HID, 16), lambda i: (0, 0)),
        ],
        out_specs=[
            pl.BlockSpec((_ROW_BLK, HID), lambda i: (i, 0)),
            pl.BlockSpec((_ROW_BLK, 16), lambda i: (i, 0)),
        ],
        out_shape=[
            jax.ShapeDtypeStruct((N_NODES, HID), jnp.float32),
            jax.ShapeDtypeStruct((N_NODES, 16), jnp.float32),
        ],
    )(out_p, den_p, bp, wp, m)


def _tc_epi_body(op_ref, dp_ref, b_ref, pinv_ref, o_ref):
    h = _combine(op_ref, dp_ref, b_ref)
    o_ref[...] = jnp.dot(h, pinv_ref[...], preferred_element_type=jnp.float32,
                         precision=lax.Precision.HIGHEST)


def _tc_epi(out_p, den_p, bp, pinv):
    return pl.pallas_call(
        _tc_epi_body,
        grid=(_GRID,),
        in_specs=[
            pl.BlockSpec((2, _ROW_BLK, HID), lambda i: (0, i, 0)),
            pl.BlockSpec((2, _ROW_BLK, 16), lambda i: (0, i, 0)),
            pl.BlockSpec((1, HID), lambda i: (0, 0)),
            pl.BlockSpec((HID, HID), lambda i: (0, 0)),
        ],
        out_specs=pl.BlockSpec((_ROW_BLK, HID), lambda i: (i, 0)),
        out_shape=jax.ShapeDtypeStruct((N_NODES, HID), jnp.float32),
    )(out_p, den_p, bp, pinv)


def _stage_alr(sid, alr_hbm, alr_tab, zrows_v):
    @pl.when(sid < NS - 1)
    def _():
        pltpu.sync_copy(alr_hbm.at[pl.ds(sid * 624, 624)],
                        alr_tab.at[pl.ds(sid * 624, 624)])

    @pl.when(sid == NS - 1)
    def _():
        pltpu.sync_copy(alr_hbm.at[pl.ds(9360, 640)],
                        alr_tab.at[pl.ds(9360, 640)])

    @pl.when(sid == 0)
    def _():
        pltpu.sync_copy(zrows_v.at[pl.ds(0, N_PAD - N_NODES)],
                        alr_tab.at[pl.ds(N_NODES, N_PAD - N_NODES)])


def _edge_logit_helpers():
    lanes = lax.iota(jnp.int32, 16)
    lo_half = lanes < 8
    swap = (lanes ^ 8)[:, None]
    gdn = lax.GatherDimensionNumbers(
        offset_dims=(), collapsed_slice_dims=(0,), start_index_map=(0,))

    def eexp_of(a_s, a_d):
        t = jnp.where(lo_half, a_s, a_d)
        e = t + lax.gather(t, swap, gdn, slice_sizes=(1,),
                           mode=lax.GatherScatterMode.PROMISE_IN_BOUNDS)
        return jnp.exp(jnp.maximum(e, e * 0.2))

    return eexp_of


def _make_sc_den(e_pad):
    epw = e_pad // NW
    ch_per_w = epw // CHUNK
    mesh = plsc.VectorSubcoreMesh(core_axis_name="c", subcore_axis_name="s")

    @functools.partial(
        pl.kernel,
        out_type=[
            jax.ShapeDtypeStruct((NC, N_PAD, 16), jnp.float32),
            jax.ShapeDtypeStruct((e_pad // 8, HID), jnp.float32),
        ],
        mesh=mesh,
        scratch_types=[
            pltpu.VMEM_SHARED((N_PAD, 16), jnp.float32),
            pltpu.VMEM_SHARED((N_PAD, 16), jnp.float32),
            pltpu.VMEM((1, CHUNK), jnp.int32),
            pltpu.VMEM((1, CHUNK), jnp.int32),
            pltpu.VMEM((CHUNK, 16), jnp.float32),
            pltpu.VMEM((CHUNK, 16), jnp.float32),
            pltpu.VMEM((CHUNK, 16), jnp.float32),
            pltpu.VMEM((CHUNK // 8, HID), jnp.float32),
            pltpu.SemaphoreType.DMA,
            pltpu.SemaphoreType.DMA,
        ],
    )
    def sc_den(src_hbm, dst_hbm, alr_hbm, den_hbm, eexp_hbm,
               den_acc, alr_tab, src_v, dst_v, a_s, a_d, eexp_v, epk_v,
               sem1, sem2):
        cid = lax.axis_index("c")
        sid = lax.axis_index("s")
        wid = cid * NS + sid
        zero16 = jnp.zeros((16,), jnp.float32)

        @pl.loop(0, CHUNK)
        def _(r):
            eexp_v[r] = zero16

        _stage_alr(sid, alr_hbm, alr_tab, eexp_v)

        base_row = sid * ROWS_PER_SUB
        off = 0
        for nrows in ROW_SPLITS:
            pltpu.sync_copy(eexp_v.at[pl.ds(0, nrows)],
                            den_acc.at[pl.ds(base_row + off, nrows)])
            off += nrows
        plsc.subcore_barrier()

        eexp_of = _edge_logit_helpers()
        ebase = wid * epw

        @pl.loop(0, ch_per_w)
        def _(j):
            eoff = ebase + j * CHUNK
            pltpu.sync_copy(src_hbm.at[pl.ds(eoff, CHUNK)], src_v.at[0])
            pltpu.sync_copy(dst_hbm.at[pl.ds(eoff, CHUNK)], dst_v.at[0])
            cp1 = pltpu.async_copy(alr_tab.at[src_v.at[0]], a_s, sem1)
            cp2 = pltpu.async_copy(alr_tab.at[dst_v.at[0]], a_d, sem2)
            cp1.wait()
            cp2.wait()

            @pl.loop(0, CHUNK)
            def _(i):
                ee = eexp_of(a_s[i], a_d[i])
                eexp_v[i] = ee
                epk_v[i // 8, pl.ds((i % 8) * 16, 16)] = ee

            pltpu.sync_copy(eexp_v, den_acc.at[dst_v.at[0]], add=True)
            erow = pl.multiple_of(eoff // 8, 8)
            pltpu.sync_copy(epk_v, eexp_hbm.at[pl.ds(erow, CHUNK // 8)])

        plsc.subcore_barrier()
        off = 0
        for nrows in ROW_SPLITS:
            pltpu.sync_copy(den_acc.at[pl.ds(base_row + off, nrows)],
                            den_hbm.at[cid, pl.ds(base_row + off, nrows)])
            off += nrows

    return sc_den


def _make_sc_out(e_pad):
    epw = e_pad // NW
    ch_per_w = epw // CHUNK
    mesh = plsc.VectorSubcoreMesh(core_axis_name="c", subcore_axis_name="s")

    @functools.partial(
        pl.kernel,
        out_type=jax.ShapeDtypeStruct((NC, N_PAD, HID), jnp.float32),
        mesh=mesh,
        scratch_types=[
            pltpu.VMEM_SHARED((N_PAD, HID), jnp.float32),
            pltpu.VMEM((1, CHUNK), jnp.int32),
            pltpu.VMEM((1, CHUNK), jnp.int32),
            pltpu.VMEM((CHUNK, HID), jnp.float32),
            pltpu.VMEM((CHUNK // 8, HID), jnp.float32),
            pltpu.VMEM((CHUNK, HID), jnp.float32),
            pltpu.SemaphoreType.DMA,
            pltpu.SemaphoreType.DMA,
        ],
    )
    def sc_out(src_hbm, dst_hbm, feat_hbm, eexp_hbm, out_hbm,
               out_acc, src_v, dst_v, feat_g, epk_v, msg_v, sem1, sem2):
        cid = lax.axis_index("c")
        sid = lax.axis_index("s")
        wid = cid * NS + sid
        zero16 = jnp.zeros((16,), jnp.float32)

        @pl.loop(0, CHUNK)
        def _(r):
            for g in range(8):
                msg_v[r, pl.ds(g * 16, 16)] = zero16

        base_row = sid * ROWS_PER_SUB
        off = 0
        for nrows in ROW_SPLITS:
            pltpu.sync_copy(msg_v.at[pl.ds(0, nrows)],
                            out_acc.at[pl.ds(base_row + off, nrows)])
            off += nrows
        plsc.subcore_barrier()

        ebase = wid * epw

        @pl.loop(0, ch_per_w)
        def _(j):
            eoff = ebase + j * CHUNK
            pltpu.sync_copy(src_hbm.at[pl.ds(eoff, CHUNK)], src_v.at[0])
            pltpu.sync_copy(dst_hbm.at[pl.ds(eoff, CHUNK)], dst_v.at[0])
            erow = pl.multiple_of(eoff // 8, 8)
            cp1 = pltpu.async_copy(
                eexp_hbm.at[pl.ds(erow, CHUNK // 8)], epk_v, sem1)
            cp2 = pltpu.async_copy(feat_hbm.at[src_v.at[0]], feat_g, sem2)
            cp1.wait()
            cp2.wait()

            @pl.loop(0, CHUNK)
            def _(i):
                ee = epk_v[i // 8, pl.ds((i % 8) * 16, 16)]
                for g in range(8):
                    msg_v[i, pl.ds(g * 16, 16)] = (
                        feat_g[i, pl.ds(g * 16, 16)] * ee)

            pltpu.sync_copy(msg_v, out_acc.at[dst_v.at[0]], add=True)

        plsc.subcore_barrier()
        off = 0
        for nrows in ROW_SPLITS:
            pltpu.sync_copy(out_acc.at[pl.ds(base_row + off, nrows)],
                            out_hbm.at[cid, pl.ds(base_row + off, nrows)])
            off += nrows

    return sc_out


def _attn_matmul_weights(attn_l, attn_r):
    alp = attn_l.reshape(HID)[SRC_COLS]
    arp = attn_r.reshape(HID)[SRC_COLS]
    hsel = jnp.asarray(HSEL)
    return jnp.concatenate([alp[:, None] * hsel, arp[:, None] * hsel], axis=1)


def kernel(x, edge_index, W1, attn_l1, attn_r1, b1, W2, attn_l2, attn_r2, b2):
    e = edge_index.shape[1]
    e_pad = ((e + NW * CHUNK - 1) // (NW * CHUNK)) * (NW * CHUNK)
    pad = e_pad - e
    src = jnp.concatenate([edge_index[0], jnp.zeros((pad,), jnp.int32)])
    dst = jnp.concatenate([edge_index[1],
                           jnp.full((pad,), N_NODES, jnp.int32)])

    cols = jnp.asarray(SRC_COLS)
    w1p = W1[:, cols]
    w2p = W2[cols][:, cols]
    m1 = _attn_matmul_weights(attn_l1, attn_r1)
    m2 = _attn_matmul_weights(attn_l2, attn_r2)
    b1p = b1[cols][None, :]
    b2p = b2[cols][None, :]
    pinv = jnp.asarray(_PINV)

    sc_den = _make_sc_den(e_pad)
    sc_out = _make_sc_out(e_pad)

    feat1, alr1 = _tc_pre(x, w1p, m1)
    den1_p, eexp1 = sc_den(src, dst, alr1)
    out1_p = sc_out(src, dst, feat1, eexp1)
    feat2, alr2 = _tc_mid(out1_p, den1_p, b1p, w2p, m2)
    den2_p, eexp2 = sc_den(src, dst, alr2)
    out2_p = sc_out(src, dst, feat2, eexp2)
    return _tc_epi(out2_p, den2_p, b2p, pinv)
